# baseline (speedup 1.0000x reference)
import functools

import jax
import jax.numpy as jnp
from jax import lax
from jax.experimental import pallas as pl
from jax.experimental.pallas import tpu as pltpu
from jax.experimental.pallas import tpu_sc as plsc

_B, _T, _N, _D = 4, 32, 257, 1024
_ROWS = _B * _T

_info = plsc.get_sparse_core_info()
_NC, _NS = _info.num_cores, _info.num_subcores
_NW = _NC * _NS
_RPW = _ROWS // _NW

_mesh = plsc.VectorSubcoreMesh(core_axis_name="c", subcore_axis_name="s")


@functools.partial(
    pl.kernel,
    mesh=_mesh,
    out_type=jax.ShapeDtypeStruct((_B, _T, _D), jnp.float32),
    scratch_types=[
        pltpu.VMEM((_RPW, _D), jnp.float32),
        pltpu.SemaphoreType.DMA,
    ],
)
def _cls_gather(img_hbm, out_hbm, buf_v, sem):
    wid = lax.axis_index("s") * _NC + lax.axis_index("c")
    b = wid // (_T // _RPW)
    t0 = (wid % (_T // _RPW)) * _RPW
    pltpu.async_copy(
        img_hbm.at[b, 0, pl.ds(t0, _RPW)], out_hbm.at[b, pl.ds(t0, _RPW)], sem
    ).wait()


def kernel(image):
    return _cls_gather(jnp.transpose(image, (0, 2, 1, 3)))


# speedup vs baseline: 1.7514x; 1.7514x over previous
import functools

import jax
import jax.numpy as jnp
from jax import lax
from jax.experimental import pallas as pl
from jax.experimental.pallas import tpu as pltpu
from jax.experimental.pallas import tpu_sc as plsc

_B, _T, _N, _D = 4, 32, 257, 1024
_ROWS = _B * _T

_info = plsc.get_sparse_core_info()
_NC, _NS = _info.num_cores, _info.num_subcores
_NW = _NC * _NS
_RPW = _ROWS // _NW

_mesh = plsc.VectorSubcoreMesh(core_axis_name="c", subcore_axis_name="s")


@functools.partial(
    pl.kernel,
    mesh=_mesh,
    out_type=jax.ShapeDtypeStruct((_B, _T, _D), jnp.float32),
    scratch_types=[
        pltpu.VMEM((_RPW, _D), jnp.float32),
        pltpu.SemaphoreType.DMA,
    ],
)
def _cls_gather(img_hbm, out_hbm, buf_v, sem):
    wid = lax.axis_index("s") * _NC + lax.axis_index("c")
    b = wid // (_T // _RPW)
    t0 = (wid % (_T // _RPW)) * _RPW
    pltpu.async_copy(img_hbm.at[b, 0, pl.ds(t0, _RPW)], buf_v, sem).wait()
    pltpu.async_copy(buf_v, out_hbm.at[b, pl.ds(t0, _RPW)], sem).wait()


def kernel(image):
    return _cls_gather(jnp.transpose(image, (0, 2, 1, 3)))


# speedup vs baseline: 1.8677x; 1.0664x over previous
import functools

import jax
import jax.numpy as jnp
from jax import lax
from jax.experimental import pallas as pl
from jax.experimental.pallas import tpu as pltpu
from jax.experimental.pallas import tpu_sc as plsc

_B, _T, _N, _D = 4, 32, 257, 1024
_BPC = _B // 2


@functools.partial(
    pl.kernel,
    mesh=plsc.ScalarSubcoreMesh(axis_name="c"),
    out_type=jax.ShapeDtypeStruct((_B, _T, _D), jnp.float32),
    scratch_types=[
        pltpu.VMEM_SHARED((_BPC, _T, _D), jnp.float32),
        pltpu.SemaphoreType.DMA,
    ],
)
def _cls_gather(img_hbm, out_hbm, buf_s, sem):
    b0 = lax.axis_index("c") * _BPC
    pltpu.async_copy(img_hbm.at[pl.ds(b0, _BPC), 0], buf_s, sem).wait()
    pltpu.async_copy(buf_s, out_hbm.at[pl.ds(b0, _BPC)], sem).wait()


def kernel(image):
    return _cls_gather(jnp.transpose(image, (0, 2, 1, 3)))


# speedup vs baseline: 10.9808x; 5.8792x over previous
import jax
import jax.numpy as jnp
from jax.experimental import pallas as pl
from jax.experimental.pallas import tpu as pltpu

_B, _T, _N, _D = 4, 32, 257, 1024


def _copy_body(img_ref, out_ref):
    out_ref[...] = img_ref[0]


_cls_slice = pl.pallas_call(
    _copy_body,
    grid=(_B,),
    in_specs=[
        pl.BlockSpec((1, 1, _T, _D), lambda i: (i, 0, 0, 0)),
    ],
    out_specs=pl.BlockSpec((1, _T, _D), lambda i: (i, 0, 0)),
    out_shape=jax.ShapeDtypeStruct((_B, _T, _D), jnp.float32),
    compiler_params=pltpu.CompilerParams(
        dimension_semantics=("arbitrary",),
    ),
)


def kernel(image):
    return _cls_slice(jnp.transpose(image, (0, 2, 1, 3)))
